# baseline (speedup 1.0000x reference)
import functools
import math

import jax
import jax.numpy as jnp
from jax import lax
from jax.experimental import pallas as pl
from jax.experimental.pallas import tpu as pltpu
from jax.experimental.pallas import tpu_sc as plsc

_N = 10000
_E = 160000
_D = 128
_H = 2
_BLK_E = 2000

_NW = 32
_EPW = _E // _NW
_CH = 40
_NCH = _EPW // _CH
_GRP = 5
_NGRP = _NCH // _GRP
_SGRP = 2


def _sc_gather(x, src2d, dst2d):
    mesh = plsc.VectorSubcoreMesh(core_axis_name="c", subcore_axis_name="s")

    @functools.partial(
        pl.kernel,
        out_type=[jax.ShapeDtypeStruct((_E, _D), jnp.float32)] * 2,
        mesh=mesh,
        scratch_types=[
            pltpu.VMEM((_NCH, _CH), jnp.int32),
            pltpu.VMEM((2, _GRP * _CH, _D), jnp.float32),
            pltpu.SemaphoreType.DMA,
            pltpu.SemaphoreType.DMA,
        ],
    )
    def k(x_hbm, src_hbm, dst_hbm, xs_hbm, xd_hbm, idx_v, buf, gsem, wsem):
        wid = lax.axis_index("c") * 16 + lax.axis_index("s")
        base = wid * _EPW

        for idx_hbm, out_hbm in ((src_hbm, xs_hbm), (dst_hbm, xd_hbm)):
            pltpu.sync_copy(idx_hbm.at[wid], idx_v)

            def out_slice(gi):
                return out_hbm.at[pl.ds(base + gi * _GRP * _CH, _GRP * _CH)]

            def group(gi, b):
                @pl.when(gi >= 2)
                def _():
                    pltpu.make_async_copy(buf.at[b], out_slice(gi - 2),
                                          wsem).wait()
                for c in range(_GRP):
                    pltpu.async_copy(x_hbm.at[idx_v.at[gi * _GRP + c]],
                                     buf.at[b, pl.ds(c * _CH, _CH)], gsem)
                for c in range(_GRP):
                    pltpu.make_async_copy(
                        x_hbm.at[idx_v.at[gi * _GRP + c]],
                        buf.at[b, pl.ds(c * _CH, _CH)], gsem).wait()
                pltpu.async_copy(buf.at[b], out_slice(gi), wsem)

            def body(i, carry):
                group(2 * i, 0)
                group(2 * i + 1, 1)
                return carry

            lax.fori_loop(0, _NGRP // 2, body, 0)
            if _NGRP % 2:
                group(_NGRP - 1, 0)
                pltpu.make_async_copy(buf.at[1], out_slice(_NGRP - 2),
                                      wsem).wait()
                pltpu.make_async_copy(buf.at[0], out_slice(_NGRP - 1),
                                      wsem).wait()
            else:
                pltpu.make_async_copy(buf.at[0], out_slice(_NGRP - 2),
                                      wsem).wait()
                pltpu.make_async_copy(buf.at[1], out_slice(_NGRP - 1),
                                      wsem).wait()

    return k(x, src2d, dst2d)


def _leaky(v):
    return jnp.where(v >= 0, v, 0.01 * v)


_NP = 10240


def _sc_scatter(m3, dst3d, zeros):
    mesh = plsc.VectorSubcoreMesh(core_axis_name="c", subcore_axis_name="s")

    @functools.partial(
        pl.kernel,
        out_type=jax.ShapeDtypeStruct((2, _NP, _D), jnp.float32),
        mesh=mesh,
        scratch_types=[
            pltpu.VMEM((_NCH, _CH), jnp.int32),
            pltpu.VMEM((2, _SGRP * _CH, _D), jnp.float32),
            pltpu.VMEM_SHARED((_NP, _D), jnp.float32),
            pltpu.SemaphoreType.DMA,
            pltpu.SemaphoreType.DMA,
        ],
    )
    def k(m3_hbm, dst_hbm, z_hbm, out_hbm, idx_v, buf, acc, rsem, ssem):
        c = lax.axis_index("c")
        s = lax.axis_index("s")
        wid = c * 16 + s
        base = wid * _EPW
        rows = pl.ds(s * 640, 640)
        pltpu.sync_copy(z_hbm.at[rows], acc.at[rows])
        pltpu.sync_copy(dst_hbm.at[wid], idx_v)
        plsc.subcore_barrier()

        def drain_adds(gi, b, n=_SGRP):
            for c2 in range(n):
                pltpu.make_async_copy(
                    buf.at[b, pl.ds(c2 * _CH, _CH)],
                    acc.at[idx_v.at[gi * _SGRP + c2]], ssem).wait()

        def group(gi, b):
            @pl.when(gi >= 2)
            def _():
                drain_adds(gi - 2, b)
            pltpu.async_copy(
                m3_hbm.at[pl.ds(base + gi * _SGRP * _CH, _SGRP * _CH)],
                buf.at[b], rsem).wait()
            for c2 in range(_SGRP):
                pltpu.async_copy(buf.at[b, pl.ds(c2 * _CH, _CH)],
                                 acc.at[idx_v.at[gi * _SGRP + c2]], ssem,
                                 add=True)

        def body(i, carry):
            group(2 * i, 0)
            group(2 * i + 1, 1)
            return carry

        nfull = _NCH // _SGRP
        lax.fori_loop(0, nfull // 2, body, 0)
        drain_adds(nfull - 2, 0)
        pltpu.async_copy(m3_hbm.at[pl.ds(base + (_NCH - 1) * _CH, _CH)],
                         buf.at[0, pl.ds(0, _CH)], rsem).wait()
        pltpu.async_copy(buf.at[0, pl.ds(0, _CH)],
                         acc.at[idx_v.at[_NCH - 1]], ssem, add=True)
        drain_adds(nfull - 1, 1)
        pltpu.make_async_copy(buf.at[0, pl.ds(0, _CH)],
                              acc.at[idx_v.at[_NCH - 1]], ssem).wait()
        plsc.subcore_barrier()
        pltpu.sync_copy(acc.at[rows], out_hbm.at[c, rows])

    return k(m3, dst3d, zeros)


def _bf(a):
    return a.astype(jnp.bfloat16)


def _dot(a, b):
    return jnp.dot(a, b, preferred_element_type=jnp.float32)


def _edge_body(xs_ref, xd_ref, ef_ref, Kv_ref, Ke_ref, Vv_ref, Ve_ref,
               luW_ref, lub_ref, lnAg_ref, lnAb_ref, msgW_ref, msgb_ref,
               msglng_ref, msglnb_ref, ccW_ref, out_ref):
    scale = 1.0 / math.sqrt(2.0 * _D)
    ones = jnp.ones((_D, _D), dtype=jnp.float32)
    xs = _bf(xs_ref[...])
    xd = _bf(xd_ref[...])
    ef = ef_ref[...]
    acc = jnp.zeros((xs.shape[0], _D), dtype=jnp.float32)
    for h in range(_H):
        Kvh = Kv_ref[h]
        q = _dot(xd, Kvh)
        k = _dot(xs, Kvh)
        v = _dot(xs, Vv_ref[h])
        KE = _dot(ef, Ke_ref[h])
        VE = _dot(ef, Ve_ref[h])
        a1 = q * k * scale
        a2 = q * KE * scale
        mu = _dot(a1 + a2, ones) * (1.0 / (2 * _D))
        ms = _dot(a1 * a1 + a2 * a2, ones) * (1.0 / (2 * _D))
        inv = lax.rsqrt(ms - mu * mu + 1e-5)
        g1 = jax.nn.sigmoid((a1 - mu) * inv * lnAg_ref[h, :_D] + lnAb_ref[h, :_D])
        g2 = jax.nn.sigmoid((a2 - mu) * inv * lnAg_ref[h, _D:] + lnAb_ref[h, _D:])
        vb = _bf(v)
        VEb = _bf(VE)
        m1a = (_dot(vb, luW_ref[h, :_D, :_D]) + _dot(VEb, luW_ref[h, _D:, :_D]) +
               lub_ref[h, :_D]) * g1
        m1b = (_dot(vb, luW_ref[h, :_D, _D:]) + _dot(VEb, luW_ref[h, _D:, _D:]) +
               lub_ref[h, _D:]) * g2
        t = (_dot(_bf(m1a), msgW_ref[h, :_D, :]) +
             _dot(_bf(m1b), msgW_ref[h, _D:, :]) + msgb_ref[h])
        tmu = _dot(t, ones) * (1.0 / _D)
        tms = _dot(t * t, ones) * (1.0 / _D)
        m2 = (t - tmu) * lax.rsqrt(tms - tmu * tmu + 1e-5) * msglng_ref[h] + msglnb_ref[h]
        m2 = _leaky(m2)
        acc = acc + _dot(_bf(m2), ccW_ref[h])
    out_ref[...] = acc


def _edge_pallas(xs, xd, ef, K_v2v, K_e2v, V_v2v, V_e2v, lu_W, lu_b,
                 lnA_g, lnA_b, msg_W, msg_b, msgln_g, msgln_b, cc_W):
    nblk = _E // _BLK_E
    eb = pl.BlockSpec((_BLK_E, _D), lambda i: (i, 0))
    full = lambda a: pl.BlockSpec(a.shape, lambda i: (0,) * a.ndim)
    bf = lambda a: a.astype(jnp.bfloat16)
    ccw3 = bf(cc_W.reshape(_H, _D, _D))
    efb = bf(ef)
    ws = (bf(K_v2v), bf(K_e2v), bf(V_v2v), bf(V_e2v), bf(lu_W), lu_b,
          lnA_g, lnA_b, bf(msg_W), msg_b, msgln_g, msgln_b, ccw3)
    return pl.pallas_call(
        _edge_body,
        grid=(nblk,),
        in_specs=[eb, eb, eb] + [full(a) for a in ws],
        out_specs=eb,
        out_shape=jax.ShapeDtypeStruct((_E, _D), jnp.float32),
    )(xs, xd, efb, *ws)


def _final_body(seg_ref, x_ref, ccb_ref, liW_ref, lib_ref, bng_ref, bnb_ref,
                out_ref):
    o = seg_ref[0, :_N, :] + seg_ref[1, :_N, :] + ccb_ref[...]
    mean = jnp.mean(o, axis=0, keepdims=True)
    oc = o - mean
    var = jnp.mean(oc * oc, axis=0, keepdims=True)
    o = oc * lax.rsqrt(var + 1e-5) * bng_ref[...] + bnb_ref[...]
    o = _leaky(o)
    out_ref[...] = o + jnp.dot(x_ref[...], liW_ref[...],
                               preferred_element_type=jnp.float32) + lib_ref[...]


def _final_pallas(seg, x, cc_b, li_W, li_b, bn_g, bn_b):
    row = lambda a: a.reshape(1, _D)
    return pl.pallas_call(
        _final_body,
        out_shape=jax.ShapeDtypeStruct((_N, _D), jnp.float32),
    )(seg, x, row(cc_b), li_W, row(li_b), row(bn_g), row(bn_b))


def kernel(x, edge_index, edge_feature, K_v2v, K_e2v, V_v2v, V_e2v, lu_W,
           lu_b, lnA_g, lnA_b, msg_W, msg_b, msgln_g, msgln_b, cc_W, cc_b,
           li_W, li_b, bn_g, bn_b):
    src = edge_index[0]
    dst = edge_index[1]
    xs, xd = _sc_gather(x, src.reshape(_NW, _NCH, _CH),
                        dst.reshape(_NW, _NCH, _CH))
    m3 = _edge_pallas(xs, xd, edge_feature, K_v2v, K_e2v, V_v2v, V_e2v,
                      lu_W, lu_b, lnA_g, lnA_b, msg_W, msg_b, msgln_g,
                      msgln_b, cc_W)
    seg = _sc_scatter(m3, dst.reshape(_NW, _NCH, _CH),
                      jnp.zeros((_NP, _D), dtype=jnp.float32))
    return _final_pallas(seg, x, cc_b, li_W, li_b, bn_g, bn_b)


# speedup vs baseline: 1.6123x; 1.6123x over previous
import functools
import math

import jax
import jax.numpy as jnp
from jax import lax
from jax.experimental import pallas as pl
from jax.experimental.pallas import tpu as pltpu
from jax.experimental.pallas import tpu_sc as plsc

_N = 10000
_E = 160000
_D = 128
_H = 2
_BLK_E = 1600

_NW = 32
_CH = 40
_GRP = 5
_SGRP = 2
_NP = 10240

_SEG_NCH = (65, 60)


def _sc_gather_seg(nch, x, src3d, dst3d):
    epw = nch * _CH
    es = _NW * epw
    ngrp = nch // _GRP
    mesh = plsc.VectorSubcoreMesh(core_axis_name="c", subcore_axis_name="s")

    @functools.partial(
        pl.kernel,
        out_type=[jax.ShapeDtypeStruct((es, _D), jnp.float32)] * 2,
        mesh=mesh,
        scratch_types=[
            pltpu.VMEM((nch, _CH), jnp.int32),
            pltpu.VMEM((2, _GRP * _CH, _D), jnp.float32),
            pltpu.SemaphoreType.DMA,
            pltpu.SemaphoreType.DMA,
        ],
    )
    def k(x_hbm, src_hbm, dst_hbm, xs_hbm, xd_hbm, idx_v, buf, gsem, wsem):
        wid = lax.axis_index("c") * 16 + lax.axis_index("s")
        base = wid * epw

        for idx_hbm, out_hbm in ((src_hbm, xs_hbm), (dst_hbm, xd_hbm)):
            pltpu.sync_copy(idx_hbm.at[wid], idx_v)

            def out_slice(gi):
                return out_hbm.at[pl.ds(base + gi * _GRP * _CH, _GRP * _CH)]

            def group(gi, b):
                @pl.when(gi >= 2)
                def _():
                    pltpu.make_async_copy(buf.at[b], out_slice(gi - 2),
                                          wsem).wait()
                for c in range(_GRP):
                    pltpu.async_copy(x_hbm.at[idx_v.at[gi * _GRP + c]],
                                     buf.at[b, pl.ds(c * _CH, _CH)], gsem)
                for c in range(_GRP):
                    pltpu.make_async_copy(
                        x_hbm.at[idx_v.at[gi * _GRP + c]],
                        buf.at[b, pl.ds(c * _CH, _CH)], gsem).wait()
                pltpu.async_copy(buf.at[b], out_slice(gi), wsem)

            def body(i, carry):
                group(2 * i, 0)
                group(2 * i + 1, 1)
                return carry

            lax.fori_loop(0, ngrp // 2, body, 0)
            if ngrp % 2:
                group(ngrp - 1, 0)
                pltpu.make_async_copy(buf.at[1], out_slice(ngrp - 2),
                                      wsem).wait()
                pltpu.make_async_copy(buf.at[0], out_slice(ngrp - 1),
                                      wsem).wait()
            else:
                pltpu.make_async_copy(buf.at[0], out_slice(ngrp - 2),
                                      wsem).wait()
                pltpu.make_async_copy(buf.at[1], out_slice(ngrp - 1),
                                      wsem).wait()

    return k(x, src3d, dst3d)


def _sc_scatter_seg(nch, m3, dst3d, zeros):
    epw = nch * _CH
    mesh = plsc.VectorSubcoreMesh(core_axis_name="c", subcore_axis_name="s")

    @functools.partial(
        pl.kernel,
        out_type=jax.ShapeDtypeStruct((2, _NP, _D), jnp.float32),
        mesh=mesh,
        scratch_types=[
            pltpu.VMEM((nch, _CH), jnp.int32),
            pltpu.VMEM((2, _SGRP * _CH, _D), jnp.float32),
            pltpu.VMEM_SHARED((_NP, _D), jnp.float32),
            pltpu.SemaphoreType.DMA,
            pltpu.SemaphoreType.DMA,
        ],
    )
    def k(m3_hbm, dst_hbm, z_hbm, out_hbm, idx_v, buf, acc, rsem, ssem):
        c = lax.axis_index("c")
        s = lax.axis_index("s")
        wid = c * 16 + s
        base = wid * epw
        rows = pl.ds(s * 640, 640)
        pltpu.sync_copy(z_hbm.at[rows], acc.at[rows])
        pltpu.sync_copy(dst_hbm.at[wid], idx_v)
        plsc.subcore_barrier()

        def drain_adds(gi, b):
            for c2 in range(_SGRP):
                pltpu.make_async_copy(
                    buf.at[b, pl.ds(c2 * _CH, _CH)],
                    acc.at[idx_v.at[gi * _SGRP + c2]], ssem).wait()

        def group(gi, b):
            @pl.when(gi >= 2)
            def _():
                drain_adds(gi - 2, b)
            pltpu.async_copy(
                m3_hbm.at[pl.ds(base + gi * _SGRP * _CH, _SGRP * _CH)],
                buf.at[b], rsem).wait()
            for c2 in range(_SGRP):
                pltpu.async_copy(buf.at[b, pl.ds(c2 * _CH, _CH)],
                                 acc.at[idx_v.at[gi * _SGRP + c2]], ssem,
                                 add=True)

        def body(i, carry):
            group(2 * i, 0)
            group(2 * i + 1, 1)
            return carry

        nfull = nch // _SGRP
        lax.fori_loop(0, nfull // 2, body, 0)
        if nch % 2:
            drain_adds(nfull - 2, 0)
            pltpu.async_copy(m3_hbm.at[pl.ds(base + (nch - 1) * _CH, _CH)],
                             buf.at[0, pl.ds(0, _CH)], rsem).wait()
            pltpu.async_copy(buf.at[0, pl.ds(0, _CH)],
                             acc.at[idx_v.at[nch - 1]], ssem, add=True)
            drain_adds(nfull - 1, 1)
            pltpu.make_async_copy(buf.at[0, pl.ds(0, _CH)],
                                  acc.at[idx_v.at[nch - 1]], ssem).wait()
        else:
            drain_adds(nfull - 2, 0)
            drain_adds(nfull - 1, 1)
        plsc.subcore_barrier()
        pltpu.sync_copy(acc.at[rows], out_hbm.at[c, rows])

    return k(m3, dst3d, zeros)


def _leaky(v):
    return jnp.where(v >= 0, v, 0.01 * v)


def _dot(a, b):
    return jnp.dot(a, b, preferred_element_type=jnp.float32)


def _edge_body(xs_ref, xd_ref, ef_ref, Kv_ref, Ke_ref, Vv_ref, Ve_ref,
               luW_ref, lub_ref, lnAg_ref, lnAb_ref, msgW_ref, msgb_ref,
               msglng_ref, msglnb_ref, ccW_ref, out_ref):
    scale = 1.0 / math.sqrt(2.0 * _D)
    xs = xs_ref[...]
    xd = xd_ref[...]
    ef = ef_ref[...]
    acc = jnp.zeros((xs.shape[0], _D), dtype=jnp.float32)
    for h in range(_H):
        Kvh = Kv_ref[h]
        q = _dot(xd, Kvh)
        k = _dot(xs, Kvh)
        v = _dot(xs, Vv_ref[h])
        KE = _dot(ef, Ke_ref[h])
        VE = _dot(ef, Ve_ref[h])
        a1 = q * k * scale
        a2 = q * KE * scale
        mu = (jnp.sum(a1, axis=-1, keepdims=True) +
              jnp.sum(a2, axis=-1, keepdims=True)) * (1.0 / (2 * _D))
        a1c = a1 - mu
        a2c = a2 - mu
        var = (jnp.sum(a1c * a1c, axis=-1, keepdims=True) +
               jnp.sum(a2c * a2c, axis=-1, keepdims=True)) * (1.0 / (2 * _D))
        inv = lax.rsqrt(var + 1e-5)
        g1 = jax.nn.sigmoid(a1c * inv * lnAg_ref[h, :_D] + lnAb_ref[h, :_D])
        g2 = jax.nn.sigmoid(a2c * inv * lnAg_ref[h, _D:] + lnAb_ref[h, _D:])
        m1a = (_dot(v, luW_ref[h, :_D, :_D]) + _dot(VE, luW_ref[h, _D:, :_D]) +
               lub_ref[h, :_D]) * g1
        m1b = (_dot(v, luW_ref[h, :_D, _D:]) + _dot(VE, luW_ref[h, _D:, _D:]) +
               lub_ref[h, _D:]) * g2
        t = (_dot(m1a, msgW_ref[h, :_D, :]) +
             _dot(m1b, msgW_ref[h, _D:, :]) + msgb_ref[h])
        tmu = jnp.mean(t, axis=-1, keepdims=True)
        tc = t - tmu
        tvar = jnp.mean(tc * tc, axis=-1, keepdims=True)
        m2 = tc * lax.rsqrt(tvar + 1e-5) * msglng_ref[h] + msglnb_ref[h]
        m2 = _leaky(m2)
        acc = acc + _dot(m2, ccW_ref[h])
    out_ref[...] = acc


def _edge_pallas(xs, xd, ef, ws):
    es = xs.shape[0]
    nblk = es // _BLK_E
    eb = pl.BlockSpec((_BLK_E, _D), lambda i: (i, 0))
    full = lambda a: pl.BlockSpec(a.shape, lambda i: (0,) * a.ndim)
    return pl.pallas_call(
        _edge_body,
        grid=(nblk,),
        in_specs=[eb, eb, eb] + [full(a) for a in ws],
        out_specs=eb,
        out_shape=jax.ShapeDtypeStruct((es, _D), jnp.float32),
    )(xs, xd, ef, *ws)


def _final_body(p0_ref, p1_ref, x_ref, ccb_ref, liW_ref, lib_ref, bng_ref,
                bnb_ref, out_ref):
    o = (p0_ref[0, :_N, :] + p0_ref[1, :_N, :] +
         p1_ref[0, :_N, :] + p1_ref[1, :_N, :] + ccb_ref[...])
    mean = jnp.mean(o, axis=0, keepdims=True)
    oc = o - mean
    var = jnp.mean(oc * oc, axis=0, keepdims=True)
    o = oc * lax.rsqrt(var + 1e-5) * bng_ref[...] + bnb_ref[...]
    o = _leaky(o)
    out_ref[...] = o + _dot(x_ref[...], liW_ref[...]) + lib_ref[...]


def _final_pallas(p0, p1, x, cc_b, li_W, li_b, bn_g, bn_b):
    row = lambda a: a.reshape(1, _D)
    return pl.pallas_call(
        _final_body,
        out_shape=jax.ShapeDtypeStruct((_N, _D), jnp.float32),
    )(p0, p1, x, row(cc_b), li_W, row(li_b), row(bn_g), row(bn_b))


def kernel(x, edge_index, edge_feature, K_v2v, K_e2v, V_v2v, V_e2v, lu_W,
           lu_b, lnA_g, lnA_b, msg_W, msg_b, msgln_g, msgln_b, cc_W, cc_b,
           li_W, li_b, bn_g, bn_b):
    src = edge_index[0]
    dst = edge_index[1]
    ws = (K_v2v, K_e2v, V_v2v, V_e2v, lu_W, lu_b, lnA_g, lnA_b,
          msg_W, msg_b, msgln_g, msgln_b, cc_W.reshape(_H, _D, _D))
    zeros = jnp.zeros((_NP, _D), dtype=jnp.float32)

    partials = []
    off = 0
    for nch in _SEG_NCH:
        es = _NW * nch * _CH
        src3d = lax.dynamic_slice(src, (off,), (es,)).reshape(_NW, nch, _CH)
        dst3d = lax.dynamic_slice(dst, (off,), (es,)).reshape(_NW, nch, _CH)
        ef = lax.dynamic_slice(edge_feature, (off, 0), (es, _D))
        xs, xd = _sc_gather_seg(nch, x, src3d, dst3d)
        m3 = _edge_pallas(xs, xd, ef, ws)
        partials.append(_sc_scatter_seg(nch, m3, dst3d, zeros))
        off += es

    return _final_pallas(partials[0], partials[1], x, cc_b, li_W, li_b,
                         bn_g, bn_b)
